# baseline (speedup 1.0000x reference)
import functools

import jax
import jax.numpy as jnp
from jax import lax
from jax.experimental import pallas as pl
from jax.experimental.pallas import tpu as pltpu
from jax.experimental.pallas import tpu_sc as plsc

B = 4096
F = 26
D = 16
ND = 4
IN_DIM = F * D
CIN = IN_DIM + D

V_ROWS = 1000000
ROWS = B * F
NW = 32
RPW = ROWS // NW
CHUNK = 128
NCHUNK = RPW // CHUNK


VC = 4096
TGRID = -(-V_ROWS // VC)


def _tr_body(x_ref, k_ref, m_ref, o_ref):
    x = x_ref[...]
    t = jnp.swapaxes(x, 0, 1)
    y2 = lax.dot_general(t, k_ref[...], (((1,), (0,)), ((), ())),
                         preferred_element_type=jnp.float32)
    y3 = y2 * m_ref[...]
    o_ref[...] = y3.reshape(VC // 8, 8, 128).sum(axis=1)


def _tc_transpose(embT):
    k = (jnp.arange(16)[:, None] == (jnp.arange(128) % 16)[None, :]).astype(jnp.float32)
    m = ((jnp.arange(VC) % 8)[:, None] == (jnp.arange(128) // 16)[None, :]).astype(jnp.float32)
    return pl.pallas_call(
        _tr_body,
        grid=(TGRID,),
        in_specs=[
            pl.BlockSpec((D, VC), lambda i: (0, i)),
            pl.BlockSpec((16, 128), lambda i: (0, 0)),
            pl.BlockSpec((VC, 128), lambda i: (0, 0)),
        ],
        out_specs=pl.BlockSpec((VC // 8, 128), lambda i: (i, 0)),
        out_shape=jax.ShapeDtypeStruct((V_ROWS * D // 128, 128), jnp.float32),
        compiler_params=pltpu.CompilerParams(dimension_semantics=("arbitrary",)),
    )(embT, k, m)


NTILE = 7813
TPW = 244


def _sc_transpose(embT, tail_final):
    mesh = plsc.VectorSubcoreMesh(core_axis_name="c", subcore_axis_name="s")

    @functools.partial(
        pl.kernel,
        mesh=mesh,
        out_type=jax.ShapeDtypeStruct((V_ROWS * D // 128, 128), jnp.float32),
        scratch_types=[
            pltpu.VMEM((2, 16, 128), jnp.float32),
            pltpu.VMEM((2, 16, 128), jnp.float32),
            pltpu.SemaphoreType.DMA,
            pltpu.SemaphoreType.DMA,
        ],
        compiler_params=pltpu.CompilerParams(needs_layout_passes=False),
    )
    def k(embT_hbm, tail_hbm, out_hbm, ibuf, obuf, semi, semo):
        wid = lax.axis_index("s") * 2 + lax.axis_index("c")
        iota16 = jax.lax.broadcasted_iota(jnp.int32, (16,), 0)

        def issue_in(t, slot):
            pltpu.async_copy(
                embT_hbm.at[pl.ds(0, 8), pl.ds(t * 128, 128)],
                ibuf.at[slot, pl.ds(0, 8), :], semi)
            pltpu.async_copy(
                embT_hbm.at[pl.ds(8, 8), pl.ds(t * 128, 128)],
                ibuf.at[slot, pl.ds(8, 8), :], semi)

        def drain_in(slot):
            pltpu.make_async_copy(
                embT_hbm.at[pl.ds(0, 8), pl.ds(0, 128)],
                ibuf.at[slot, pl.ds(0, 8), :], semi).wait()
            pltpu.make_async_copy(
                embT_hbm.at[pl.ds(8, 8), pl.ds(0, 128)],
                ibuf.at[slot, pl.ds(8, 8), :], semi).wait()

        def extract(slot, nrows):
            sidx = jnp.full((16,), slot, jnp.int32)
            for r in range(nrows):
                for s in range(8):
                    col = jnp.full((16,), 8 * r + s, jnp.int32)
                    piece = plsc.load_gather(ibuf, [sidx, iota16, col])
                    obuf[slot, r, pl.ds(16 * s, 16)] = piece

        def issue_out(t, slot):
            pltpu.async_copy(obuf.at[slot], out_hbm.at[pl.ds(t * 16, 16)], semo)

        def drain_out(slot):
            pltpu.make_async_copy(
                obuf.at[slot], out_hbm.at[pl.ds(0, 16)], semo).wait()

        issue_in(wid, 0)

        def body(j, _):
            t0 = wid + 64 * j
            drain_in(0)
            issue_in(t0 + 32, 1)

            @pl.when(j >= 1)
            def _():
                drain_out(0)

            extract(0, 16)
            issue_out(t0, 0)
            drain_in(1)

            @pl.when(j + 1 < TPW // 2)
            def _():
                issue_in(t0 + 64, 0)

            @pl.when(j >= 1)
            def _():
                drain_out(1)

            extract(1, 16)
            issue_out(t0 + 32, 1)
            return 0

        lax.fori_loop(0, TPW // 2, body, 0)
        drain_out(0)
        drain_out(1)

        @pl.when(wid < 4)
        def _():
            t = 7808 + wid
            pltpu.sync_copy(embT_hbm.at[pl.ds(0, 8), pl.ds(t * 128, 128)],
                            ibuf.at[0, pl.ds(0, 8), :])
            pltpu.sync_copy(embT_hbm.at[pl.ds(8, 8), pl.ds(t * 128, 128)],
                            ibuf.at[0, pl.ds(8, 8), :])
            extract(0, 16)
            pltpu.sync_copy(obuf.at[0], out_hbm.at[pl.ds(t * 16, 16)])

        @pl.when(wid == 4)
        def _():
            pltpu.sync_copy(tail_hbm, out_hbm.at[pl.ds(124992, 8)])

    return k(embT, tail_final)


def _sc_gather(table, ids3):
    mesh = plsc.VectorSubcoreMesh(core_axis_name="c", subcore_axis_name="s")

    @functools.partial(
        pl.kernel,
        mesh=mesh,
        out_type=jax.ShapeDtypeStruct((ROWS, D), jnp.float32),
        scratch_types=[
            pltpu.VMEM((NCHUNK, CHUNK), jnp.int32),
            pltpu.VMEM((RPW, D), jnp.float32),
            pltpu.SemaphoreType.DMA,
        ],
        compiler_params=pltpu.CompilerParams(use_tc_tiling_on_sc=False),
    )
    def k(table_hbm, ids_hbm, out_hbm, idx_v, rows_v, sem):
        wid = lax.axis_index("s") * 2 + lax.axis_index("c")
        base = wid * RPW
        pltpu.sync_copy(ids_hbm.at[wid], idx_v)
        copies = []
        for j in range(NCHUNK):
            copies.append(
                pltpu.async_copy(
                    table_hbm.at[idx_v.at[j]],
                    rows_v.at[pl.ds(j * CHUNK, CHUNK)],
                    sem,
                )
            )
        for c in copies:
            c.wait()
        pltpu.sync_copy(rows_v, out_hbm.at[pl.ds(base, RPW)])

    return k(table, ids3)


BLK = 512
GRID = B // BLK
DOT_PREC = lax.Precision.DEFAULT


def _tc_body(
    x_ref, oh_ref,
    dom_emb_ref, pnw_ref, pnb_ref,
    cwm_ref, cwd_ref, cbm_ref, cbd_ref,
    c1m_ref, c1d_ref, c1b_ref, c2_ref, c2b_ref, c3_ref, c3b_ref,
    d1m_ref, d1d_ref, d1b_ref, d2_ref, d2b_ref, d3_ref, d3b_ref,
    f1_ref, f1b_ref, f2_ref, f2b_ref,
    a1_ref, a1b_ref, a2_ref, a2b_ref,
    out_ref,
):
    f32 = jnp.float32
    x = x_ref[...]
    oh = oh_ref[...]

    mean = jnp.mean(x, axis=1, keepdims=True)
    xc = x - mean
    var = jnp.mean(xc * xc, axis=1, keepdims=True)
    xn = xc * lax.rsqrt(var + 1e-5)

    pnw = jnp.dot(oh, pnw_ref[...], preferred_element_type=f32, precision=DOT_PREC)
    pnb = jnp.dot(oh, pnb_ref[...], preferred_element_type=f32, precision=DOT_PREC)
    xn = xn * pnw + pnb
    dom = jnp.dot(oh, dom_emb_ref[...], preferred_element_type=f32, precision=DOT_PREC)

    x0m, x0d = xn, dom
    xm, xd = xn, dom
    for i in range(3):
        proj = (
            jnp.sum(xm * cwm_ref[i][None, :], axis=1, keepdims=True)
            + jnp.sum(xd * cwd_ref[i][None, :], axis=1, keepdims=True)
        )
        xm = x0m * proj + cbm_ref[i][None, :] + xm
        xd = x0d * proj + cbd_ref[i][None, :] + xd

    h = jax.nn.relu(
        jnp.dot(xm, c1m_ref[...], preferred_element_type=f32, precision=DOT_PREC)
        + jnp.dot(xd, c1d_ref[...], preferred_element_type=f32, precision=DOT_PREC)
        + c1b_ref[...]
    )
    h = jax.nn.relu(jnp.dot(h, c2_ref[...], preferred_element_type=f32, precision=DOT_PREC) + c2b_ref[...])
    center = jnp.dot(h, c3_ref[...], preferred_element_type=f32, precision=DOT_PREC) + c3b_ref[...]

    domout = jnp.zeros((BLK, 128), f32)
    for d in range(ND):
        h1 = jax.nn.relu(
            jnp.dot(xm, d1m_ref[d], preferred_element_type=f32, precision=DOT_PREC)
            + jnp.dot(xd, d1d_ref[d], preferred_element_type=f32, precision=DOT_PREC)
            + d1b_ref[d][None, :]
        )
        h2 = jax.nn.relu(jnp.dot(h1, d2_ref[d], preferred_element_type=f32, precision=DOT_PREC) + d2b_ref[d][None, :])
        h3 = jnp.dot(h2, d3_ref[d], preferred_element_type=f32, precision=DOT_PREC) + d3b_ref[d][None, :]
        domout = domout + oh[:, d:d + 1] * h3

    fused = center * jnp.tanh(domout)
    h = jax.nn.relu(jnp.dot(fused, f1_ref[...], preferred_element_type=f32, precision=DOT_PREC) + f1b_ref[...])
    logit = jnp.dot(h, f2_ref[...], preferred_element_type=f32, precision=DOT_PREC) + f2b_ref[...]

    ha = jax.nn.relu(jnp.dot(dom, a1_ref[...], preferred_element_type=f32, precision=DOT_PREC) + a1b_ref[...])
    aux = jnp.dot(ha, a2_ref[...], preferred_element_type=f32, precision=DOT_PREC) + a2b_ref[...]

    out_ref[...] = jax.nn.sigmoid(logit + aux)


def _const_spec(shape):
    rank = len(shape)
    return pl.BlockSpec(shape, lambda i: (0,) * rank)


def _tc_forward(shared, oh, weights, interpret=False):
    in_specs = [
        pl.BlockSpec((BLK, IN_DIM), lambda i: (i, 0)),
        pl.BlockSpec((BLK, ND), lambda i: (i, 0)),
    ] + [_const_spec(w.shape) for w in weights]
    return pl.pallas_call(
        _tc_body,
        grid=(GRID,),
        in_specs=in_specs,
        out_specs=pl.BlockSpec((BLK, 1), lambda i: (i, 0)),
        out_shape=jax.ShapeDtypeStruct((B, 1), jnp.float32),
        compiler_params=pltpu.CompilerParams(
            dimension_semantics=("arbitrary",),
        ),
        interpret=interpret,
    )(shared, oh, *weights)


def _prep_weights(params):
    cw = params["cross_w"]
    cb = params["cross_b"]
    c1w, c1b = params["center"][0]
    c2w, c2b = params["center"][1]
    c3w, c3b = params["center"][2]
    f1w, f1b = params["final"][0]
    f2w, f2b = params["final"][1]
    a1w, a1b = params["aux"][0]
    a2w, a2b = params["aux"][1]
    row = lambda v: v.reshape(1, -1)
    return [
        params["dom_emb"],
        params["pn_w"], params["pn_b"],
        cw[:, :IN_DIM], cw[:, IN_DIM:],
        cb[:, :IN_DIM], cb[:, IN_DIM:],
        c1w[:IN_DIM], c1w[IN_DIM:], row(c1b),
        c2w, row(c2b), c3w, row(c3b),
        params["dW1"][:, :IN_DIM], params["dW1"][:, IN_DIM:], params["db1"],
        params["dW2"], params["db2"], params["dW3"], params["db3"],
        f1w, row(f1b), f2w, row(f2b),
        a1w, row(a1b), a2w, row(a2b),
    ]


def kernel(sparse_ids, domain_ids, params):
    ids3 = sparse_ids.reshape(NW, NCHUNK, CHUNK)
    tail_final = params["emb"][999936:].reshape(8, 128)
    table128 = _sc_transpose(params["emb"].T, tail_final)
    table = table128.reshape(V_ROWS, D)
    rows = _sc_gather(table, ids3)
    shared = rows.reshape(B, IN_DIM)
    oh = jax.nn.one_hot(domain_ids, ND, dtype=jnp.float32)
    weights = _prep_weights(params)
    return _tc_forward(shared, oh, weights)


# speedup vs baseline: 1.5289x; 1.5289x over previous
import functools

import jax
import jax.numpy as jnp
from jax import lax
from jax.experimental import pallas as pl
from jax.experimental.pallas import tpu as pltpu
from jax.experimental.pallas import tpu_sc as plsc

B = 4096
F = 26
D = 16
ND = 4
IN_DIM = F * D
CIN = IN_DIM + D

V_ROWS = 1000000
ROWS = B * F
NW = 32
RPW = ROWS // NW
CHUNK = 128
NCHUNK = RPW // CHUNK


VC = 4096
TGRID = -(-V_ROWS // VC)


def _tr_body(x_ref, k_ref, m_ref, o_ref):
    x = x_ref[...]
    t = jnp.swapaxes(x, 0, 1)
    y2 = lax.dot_general(t, k_ref[...], (((1,), (0,)), ((), ())),
                         preferred_element_type=jnp.float32)
    y3 = y2 * m_ref[...]
    o_ref[...] = y3.reshape(VC // 8, 8, 128).sum(axis=1)


def _tc_transpose(embT):
    k = (jnp.arange(16)[:, None] == (jnp.arange(128) % 16)[None, :]).astype(jnp.float32)
    m = ((jnp.arange(VC) % 8)[:, None] == (jnp.arange(128) // 16)[None, :]).astype(jnp.float32)
    return pl.pallas_call(
        _tr_body,
        grid=(TGRID,),
        in_specs=[
            pl.BlockSpec((D, VC), lambda i: (0, i)),
            pl.BlockSpec((16, 128), lambda i: (0, 0)),
            pl.BlockSpec((VC, 128), lambda i: (0, 0)),
        ],
        out_specs=pl.BlockSpec((VC // 8, 128), lambda i: (i, 0)),
        out_shape=jax.ShapeDtypeStruct((V_ROWS * D // 128, 128), jnp.float32),
        compiler_params=pltpu.CompilerParams(dimension_semantics=("arbitrary",)),
    )(embT, k, m)


NTILE = 7813
TPW = 244


def _sc_transpose(embT, tail_final):
    mesh = plsc.VectorSubcoreMesh(core_axis_name="c", subcore_axis_name="s")

    @functools.partial(
        pl.kernel,
        mesh=mesh,
        out_type=jax.ShapeDtypeStruct((V_ROWS * D // 128, 128), jnp.float32),
        scratch_types=[
            pltpu.VMEM((2, 16, 128), jnp.float32),
            pltpu.VMEM((2, 16, 128), jnp.float32),
            pltpu.SemaphoreType.DMA,
            pltpu.SemaphoreType.DMA,
        ],
        compiler_params=pltpu.CompilerParams(needs_layout_passes=False),
    )
    def k(embT_hbm, tail_hbm, out_hbm, ibuf, obuf, semi, semo):
        wid = lax.axis_index("s") * 2 + lax.axis_index("c")
        iota16 = jax.lax.broadcasted_iota(jnp.int32, (16,), 0)

        def issue_in(t, slot):
            pltpu.async_copy(
                embT_hbm.at[pl.ds(0, 8), pl.ds(t * 128, 128)],
                ibuf.at[slot, pl.ds(0, 8), :], semi)
            pltpu.async_copy(
                embT_hbm.at[pl.ds(8, 8), pl.ds(t * 128, 128)],
                ibuf.at[slot, pl.ds(8, 8), :], semi)

        def drain_in(slot):
            pltpu.make_async_copy(
                embT_hbm.at[pl.ds(0, 8), pl.ds(0, 128)],
                ibuf.at[slot, pl.ds(0, 8), :], semi).wait()
            pltpu.make_async_copy(
                embT_hbm.at[pl.ds(8, 8), pl.ds(0, 128)],
                ibuf.at[slot, pl.ds(8, 8), :], semi).wait()

        def extract(slot, nrows):
            sidx = jnp.full((16,), slot, jnp.int32)
            for r in range(nrows):
                for s in range(8):
                    col = jnp.full((16,), 8 * r + s, jnp.int32)
                    piece = plsc.load_gather(ibuf, [sidx, iota16, col])
                    obuf[slot, r, pl.ds(16 * s, 16)] = piece

        def issue_out(t, slot):
            pltpu.async_copy(obuf.at[slot], out_hbm.at[pl.ds(t * 16, 16)], semo)

        def drain_out(slot):
            pltpu.make_async_copy(
                obuf.at[slot], out_hbm.at[pl.ds(0, 16)], semo).wait()

        issue_in(wid, 0)

        def body(j, _):
            t0 = wid + 64 * j
            drain_in(0)
            issue_in(t0 + 32, 1)

            @pl.when(j >= 1)
            def _():
                drain_out(0)

            extract(0, 16)
            issue_out(t0, 0)
            drain_in(1)

            @pl.when(j + 1 < TPW // 2)
            def _():
                issue_in(t0 + 64, 0)

            @pl.when(j >= 1)
            def _():
                drain_out(1)

            extract(1, 16)
            issue_out(t0 + 32, 1)
            return 0

        lax.fori_loop(0, TPW // 2, body, 0)
        drain_out(0)
        drain_out(1)

        @pl.when(wid < 4)
        def _():
            t = 7808 + wid
            pltpu.sync_copy(embT_hbm.at[pl.ds(0, 8), pl.ds(t * 128, 128)],
                            ibuf.at[0, pl.ds(0, 8), :])
            pltpu.sync_copy(embT_hbm.at[pl.ds(8, 8), pl.ds(t * 128, 128)],
                            ibuf.at[0, pl.ds(8, 8), :])
            extract(0, 16)
            pltpu.sync_copy(obuf.at[0], out_hbm.at[pl.ds(t * 16, 16)])

        @pl.when(wid == 4)
        def _():
            pltpu.sync_copy(tail_hbm, out_hbm.at[pl.ds(124992, 8)])

    return k(embT, tail_final)


def _sc_gather(table, ids3):
    mesh = plsc.VectorSubcoreMesh(core_axis_name="c", subcore_axis_name="s")

    @functools.partial(
        pl.kernel,
        mesh=mesh,
        out_type=jax.ShapeDtypeStruct((ROWS, D), jnp.float32),
        scratch_types=[
            pltpu.VMEM((NCHUNK, CHUNK), jnp.int32),
            pltpu.VMEM((RPW, D), jnp.float32),
            pltpu.SemaphoreType.DMA,
        ],
        compiler_params=pltpu.CompilerParams(use_tc_tiling_on_sc=False),
    )
    def k(table_hbm, ids_hbm, out_hbm, idx_v, rows_v, sem):
        wid = lax.axis_index("s") * 2 + lax.axis_index("c")
        base = wid * RPW
        pltpu.sync_copy(ids_hbm.at[wid], idx_v)
        copies = []
        for j in range(NCHUNK):
            copies.append(
                pltpu.async_copy(
                    table_hbm.at[idx_v.at[j]],
                    rows_v.at[pl.ds(j * CHUNK, CHUNK)],
                    sem,
                )
            )
        for c in copies:
            c.wait()
        pltpu.sync_copy(rows_v, out_hbm.at[pl.ds(base, RPW)])

    return k(table, ids3)


BLK = 512
GRID = B // BLK
DOT_PREC = lax.Precision.DEFAULT


def _tc_body(
    x_ref, oh_ref,
    dom_emb_ref, pnw_ref, pnb_ref,
    cwm_ref, cwd_ref, cbm_ref, cbd_ref,
    c1m_ref, c1d_ref, c1b_ref, c2_ref, c2b_ref, c3_ref, c3b_ref,
    d1m_ref, d1d_ref, d1b_ref, d2_ref, d2b_ref, d3_ref, d3b_ref,
    f1_ref, f1b_ref, f2_ref, f2b_ref,
    a1_ref, a1b_ref, a2_ref, a2b_ref,
    out_ref,
):
    f32 = jnp.float32
    x = x_ref[...]
    oh = oh_ref[...]

    mean = jnp.mean(x, axis=1, keepdims=True)
    xc = x - mean
    var = jnp.mean(xc * xc, axis=1, keepdims=True)
    xn = xc * lax.rsqrt(var + 1e-5)

    pnw = jnp.dot(oh, pnw_ref[...], preferred_element_type=f32, precision=DOT_PREC)
    pnb = jnp.dot(oh, pnb_ref[...], preferred_element_type=f32, precision=DOT_PREC)
    xn = xn * pnw + pnb
    dom = jnp.dot(oh, dom_emb_ref[...], preferred_element_type=f32, precision=DOT_PREC)

    x0m, x0d = xn, dom
    xm, xd = xn, dom
    for i in range(3):
        proj = (
            jnp.sum(xm * cwm_ref[i][None, :], axis=1, keepdims=True)
            + jnp.sum(xd * cwd_ref[i][None, :], axis=1, keepdims=True)
        )
        xm = x0m * proj + cbm_ref[i][None, :] + xm
        xd = x0d * proj + cbd_ref[i][None, :] + xd

    bf16 = jnp.bfloat16
    xm16 = xm.astype(bf16)
    xd16 = xd.astype(bf16)
    h = jax.nn.relu(
        jnp.dot(xm16, c1m_ref[...], preferred_element_type=f32, precision=DOT_PREC)
        + jnp.dot(xd16, c1d_ref[...], preferred_element_type=f32, precision=DOT_PREC)
        + c1b_ref[...]
    )
    h = jax.nn.relu(jnp.dot(h.astype(bf16), c2_ref[...], preferred_element_type=f32, precision=DOT_PREC) + c2b_ref[...])
    center = jnp.dot(h.astype(bf16), c3_ref[...], preferred_element_type=f32, precision=DOT_PREC) + c3b_ref[...]

    domout = jnp.zeros((BLK, 128), f32)
    for d in range(ND):
        h1 = jax.nn.relu(
            jnp.dot(xm16, d1m_ref[d], preferred_element_type=f32, precision=DOT_PREC)
            + jnp.dot(xd16, d1d_ref[d], preferred_element_type=f32, precision=DOT_PREC)
            + d1b_ref[d][None, :]
        )
        h2 = jax.nn.relu(jnp.dot(h1.astype(bf16), d2_ref[d], preferred_element_type=f32, precision=DOT_PREC) + d2b_ref[d][None, :])
        h3 = jnp.dot(h2.astype(bf16), d3_ref[d], preferred_element_type=f32, precision=DOT_PREC) + d3b_ref[d][None, :]
        domout = domout + oh[:, d:d + 1] * h3

    fused = center * jnp.tanh(domout)
    h = jax.nn.relu(jnp.dot(fused, f1_ref[...], preferred_element_type=f32, precision=DOT_PREC) + f1b_ref[...])
    logit = jnp.dot(h, f2_ref[...], preferred_element_type=f32, precision=DOT_PREC) + f2b_ref[...]

    ha = jax.nn.relu(jnp.dot(dom, a1_ref[...], preferred_element_type=f32, precision=DOT_PREC) + a1b_ref[...])
    aux = jnp.dot(ha, a2_ref[...], preferred_element_type=f32, precision=DOT_PREC) + a2b_ref[...]

    out_ref[...] = jax.nn.sigmoid(logit + aux)


def _const_spec(shape):
    rank = len(shape)
    return pl.BlockSpec(shape, lambda i: (0,) * rank)


def _tc_forward(shared, oh, weights, interpret=False):
    in_specs = [
        pl.BlockSpec((BLK, IN_DIM), lambda i: (i, 0)),
        pl.BlockSpec((BLK, ND), lambda i: (i, 0)),
    ] + [_const_spec(w.shape) for w in weights]
    return pl.pallas_call(
        _tc_body,
        grid=(GRID,),
        in_specs=in_specs,
        out_specs=pl.BlockSpec((BLK, 1), lambda i: (i, 0)),
        out_shape=jax.ShapeDtypeStruct((B, 1), jnp.float32),
        compiler_params=pltpu.CompilerParams(
            dimension_semantics=("arbitrary",),
        ),
        interpret=interpret,
    )(shared, oh, *weights)


def _prep_weights(params):
    cw = params["cross_w"]
    cb = params["cross_b"]
    c1w, c1b = params["center"][0]
    c2w, c2b = params["center"][1]
    c3w, c3b = params["center"][2]
    f1w, f1b = params["final"][0]
    f2w, f2b = params["final"][1]
    a1w, a1b = params["aux"][0]
    a2w, a2b = params["aux"][1]
    row = lambda v: v.reshape(1, -1)
    return [
        params["dom_emb"],
        params["pn_w"], params["pn_b"],
        cw[:, :IN_DIM], cw[:, IN_DIM:],
        cb[:, :IN_DIM], cb[:, IN_DIM:],
        c1w[:IN_DIM].astype(jnp.bfloat16), c1w[IN_DIM:].astype(jnp.bfloat16), row(c1b),
        c2w.astype(jnp.bfloat16), row(c2b), c3w.astype(jnp.bfloat16), row(c3b),
        params["dW1"][:, :IN_DIM].astype(jnp.bfloat16),
        params["dW1"][:, IN_DIM:].astype(jnp.bfloat16), params["db1"],
        params["dW2"].astype(jnp.bfloat16), params["db2"],
        params["dW3"].astype(jnp.bfloat16), params["db3"],
        f1w, row(f1b), f2w, row(f2b),
        a1w, row(a1b), a2w, row(a2b),
    ]


def kernel(sparse_ids, domain_ids, params):
    ids3 = sparse_ids.reshape(NW, NCHUNK, CHUNK)
    table128 = _tc_transpose(params["emb"].T)
    table = table128.reshape(V_ROWS, D)
    rows = _sc_gather(table, ids3)
    shared = rows.reshape(B, IN_DIM)
    oh = jax.nn.one_hot(domain_ids, ND, dtype=jnp.float32)
    weights = _prep_weights(params)
    return _tc_forward(shared, oh, weights)


# speedup vs baseline: 1.8879x; 1.2349x over previous
import functools

import jax
import jax.numpy as jnp
from jax import lax
from jax.experimental import pallas as pl
from jax.experimental.pallas import tpu as pltpu
from jax.experimental.pallas import tpu_sc as plsc

B = 4096
F = 26
D = 16
ND = 4
IN_DIM = F * D
CIN = IN_DIM + D

V_ROWS = 1000000
ROWS = B * F
NW = 32
RPW = ROWS // NW
CHUNK = 128
NCHUNK = RPW // CHUNK


VC = 8192
TGRID = -(-V_ROWS // VC)


def _tr_body(x_ref, k_ref, m_ref, o_ref):
    x = x_ref[...]
    t = jnp.swapaxes(x, 0, 1)
    y2 = lax.dot_general(t, k_ref[...], (((1,), (0,)), ((), ())),
                         preferred_element_type=jnp.float32)
    y3 = y2 * m_ref[...]
    o_ref[...] = y3.reshape(VC // 8, 8, 128).sum(axis=1)


def _tc_transpose(embT):
    k = (jnp.arange(16)[:, None] == (jnp.arange(128) % 16)[None, :]).astype(jnp.float32)
    m = ((jnp.arange(VC) % 8)[:, None] == (jnp.arange(128) // 16)[None, :]).astype(jnp.float32)
    return pl.pallas_call(
        _tr_body,
        grid=(TGRID,),
        in_specs=[
            pl.BlockSpec((D, VC), lambda i: (0, i)),
            pl.BlockSpec((16, 128), lambda i: (0, 0)),
            pl.BlockSpec((VC, 128), lambda i: (0, 0)),
        ],
        out_specs=pl.BlockSpec((VC // 8, 128), lambda i: (i, 0)),
        out_shape=jax.ShapeDtypeStruct((V_ROWS * D // 128, 128), jnp.float32),
        compiler_params=pltpu.CompilerParams(dimension_semantics=("arbitrary",)),
    )(embT, k, m)


NTILE = 7813
TPW = 244


def _sc_transpose(embT, tail_final):
    mesh = plsc.VectorSubcoreMesh(core_axis_name="c", subcore_axis_name="s")

    @functools.partial(
        pl.kernel,
        mesh=mesh,
        out_type=jax.ShapeDtypeStruct((V_ROWS * D // 128, 128), jnp.float32),
        scratch_types=[
            pltpu.VMEM((2, 16, 128), jnp.float32),
            pltpu.VMEM((2, 16, 128), jnp.float32),
            pltpu.SemaphoreType.DMA,
            pltpu.SemaphoreType.DMA,
        ],
        compiler_params=pltpu.CompilerParams(needs_layout_passes=False),
    )
    def k(embT_hbm, tail_hbm, out_hbm, ibuf, obuf, semi, semo):
        wid = lax.axis_index("s") * 2 + lax.axis_index("c")
        iota16 = jax.lax.broadcasted_iota(jnp.int32, (16,), 0)

        def issue_in(t, slot):
            pltpu.async_copy(
                embT_hbm.at[pl.ds(0, 8), pl.ds(t * 128, 128)],
                ibuf.at[slot, pl.ds(0, 8), :], semi)
            pltpu.async_copy(
                embT_hbm.at[pl.ds(8, 8), pl.ds(t * 128, 128)],
                ibuf.at[slot, pl.ds(8, 8), :], semi)

        def drain_in(slot):
            pltpu.make_async_copy(
                embT_hbm.at[pl.ds(0, 8), pl.ds(0, 128)],
                ibuf.at[slot, pl.ds(0, 8), :], semi).wait()
            pltpu.make_async_copy(
                embT_hbm.at[pl.ds(8, 8), pl.ds(0, 128)],
                ibuf.at[slot, pl.ds(8, 8), :], semi).wait()

        def extract(slot, nrows):
            sidx = jnp.full((16,), slot, jnp.int32)
            for r in range(nrows):
                for s in range(8):
                    col = jnp.full((16,), 8 * r + s, jnp.int32)
                    piece = plsc.load_gather(ibuf, [sidx, iota16, col])
                    obuf[slot, r, pl.ds(16 * s, 16)] = piece

        def issue_out(t, slot):
            pltpu.async_copy(obuf.at[slot], out_hbm.at[pl.ds(t * 16, 16)], semo)

        def drain_out(slot):
            pltpu.make_async_copy(
                obuf.at[slot], out_hbm.at[pl.ds(0, 16)], semo).wait()

        issue_in(wid, 0)

        def body(j, _):
            t0 = wid + 64 * j
            drain_in(0)
            issue_in(t0 + 32, 1)

            @pl.when(j >= 1)
            def _():
                drain_out(0)

            extract(0, 16)
            issue_out(t0, 0)
            drain_in(1)

            @pl.when(j + 1 < TPW // 2)
            def _():
                issue_in(t0 + 64, 0)

            @pl.when(j >= 1)
            def _():
                drain_out(1)

            extract(1, 16)
            issue_out(t0 + 32, 1)
            return 0

        lax.fori_loop(0, TPW // 2, body, 0)
        drain_out(0)
        drain_out(1)

        @pl.when(wid < 4)
        def _():
            t = 7808 + wid
            pltpu.sync_copy(embT_hbm.at[pl.ds(0, 8), pl.ds(t * 128, 128)],
                            ibuf.at[0, pl.ds(0, 8), :])
            pltpu.sync_copy(embT_hbm.at[pl.ds(8, 8), pl.ds(t * 128, 128)],
                            ibuf.at[0, pl.ds(8, 8), :])
            extract(0, 16)
            pltpu.sync_copy(obuf.at[0], out_hbm.at[pl.ds(t * 16, 16)])

        @pl.when(wid == 4)
        def _():
            pltpu.sync_copy(tail_hbm, out_hbm.at[pl.ds(124992, 8)])

    return k(embT, tail_final)


def _sc_gather(table, ids3):
    mesh = plsc.VectorSubcoreMesh(core_axis_name="c", subcore_axis_name="s")

    @functools.partial(
        pl.kernel,
        mesh=mesh,
        out_type=jax.ShapeDtypeStruct((ROWS, D), jnp.float32),
        scratch_types=[
            pltpu.VMEM((NCHUNK, CHUNK), jnp.int32),
            pltpu.VMEM((RPW, D), jnp.float32),
            pltpu.SemaphoreType.DMA,
        ],
        compiler_params=pltpu.CompilerParams(use_tc_tiling_on_sc=False),
    )
    def k(table_hbm, ids_hbm, out_hbm, idx_v, rows_v, sem):
        wid = lax.axis_index("s") * 2 + lax.axis_index("c")
        base = wid * RPW
        pltpu.sync_copy(ids_hbm.at[wid], idx_v)
        copies = []
        for j in range(NCHUNK):
            copies.append(
                pltpu.async_copy(
                    table_hbm.at[idx_v.at[j]],
                    rows_v.at[pl.ds(j * CHUNK, CHUNK)],
                    sem,
                )
            )
        for c in copies:
            c.wait()
        pltpu.sync_copy(rows_v, out_hbm.at[pl.ds(base, RPW)])

    return k(table, ids3)


BLK = 512
GRID = B // BLK
DOT_PREC = lax.Precision.DEFAULT


def _tc_body(
    x_ref, oh_ref,
    dom_emb_ref, pnw_ref, pnb_ref,
    cwm_ref, cwd_ref, cbm_ref, cbd_ref,
    c1m_ref, c1d_ref, c1b_ref, c2_ref, c2b_ref, c3_ref, c3b_ref,
    d1m_ref, d1d_ref, d1b_ref, d2_ref, d2b_ref, d3_ref, d3b_ref,
    f1_ref, f1b_ref, f2_ref, f2b_ref,
    a1_ref, a1b_ref, a2_ref, a2b_ref,
    out_ref,
):
    f32 = jnp.float32
    x = x_ref[...]
    oh = oh_ref[...]

    mean = jnp.mean(x, axis=1, keepdims=True)
    xc = x - mean
    var = jnp.mean(xc * xc, axis=1, keepdims=True)
    xn = xc * lax.rsqrt(var + 1e-5)

    pnw = jnp.dot(oh, pnw_ref[...], preferred_element_type=f32, precision=DOT_PREC)
    pnb = jnp.dot(oh, pnb_ref[...], preferred_element_type=f32, precision=DOT_PREC)
    xn = xn * pnw + pnb
    dom = jnp.dot(oh, dom_emb_ref[...], preferred_element_type=f32, precision=DOT_PREC)

    x0m, x0d = xn, dom
    xm, xd = xn, dom
    for i in range(3):
        proj = (
            jnp.sum(xm * cwm_ref[i][None, :], axis=1, keepdims=True)
            + jnp.sum(xd * cwd_ref[i][None, :], axis=1, keepdims=True)
        )
        xm = x0m * proj + cbm_ref[i][None, :] + xm
        xd = x0d * proj + cbd_ref[i][None, :] + xd

    h = jax.nn.relu(
        jnp.dot(xm, c1m_ref[...], preferred_element_type=f32, precision=DOT_PREC)
        + jnp.dot(xd, c1d_ref[...], preferred_element_type=f32, precision=DOT_PREC)
        + c1b_ref[...]
    )
    h = jax.nn.relu(jnp.dot(h, c2_ref[...], preferred_element_type=f32, precision=DOT_PREC) + c2b_ref[...])
    center = jnp.dot(h, c3_ref[...], preferred_element_type=f32, precision=DOT_PREC) + c3b_ref[...]

    domout = jnp.zeros((BLK, 128), f32)
    for d in range(ND):
        h1 = jax.nn.relu(
            jnp.dot(xm, d1m_ref[d], preferred_element_type=f32, precision=DOT_PREC)
            + jnp.dot(xd, d1d_ref[d], preferred_element_type=f32, precision=DOT_PREC)
            + d1b_ref[d][None, :]
        )
        h2 = jax.nn.relu(jnp.dot(h1, d2_ref[d], preferred_element_type=f32, precision=DOT_PREC) + d2b_ref[d][None, :])
        h3 = jnp.dot(h2, d3_ref[d], preferred_element_type=f32, precision=DOT_PREC) + d3b_ref[d][None, :]
        domout = domout + oh[:, d:d + 1] * h3

    fused = center * jnp.tanh(domout)
    h = jax.nn.relu(jnp.dot(fused, f1_ref[...], preferred_element_type=f32, precision=DOT_PREC) + f1b_ref[...])
    logit = jnp.dot(h, f2_ref[...], preferred_element_type=f32, precision=DOT_PREC) + f2b_ref[...]

    ha = jax.nn.relu(jnp.dot(dom, a1_ref[...], preferred_element_type=f32, precision=DOT_PREC) + a1b_ref[...])
    aux = jnp.dot(ha, a2_ref[...], preferred_element_type=f32, precision=DOT_PREC) + a2b_ref[...]

    out_ref[...] = jax.nn.sigmoid(logit + aux)


def _const_spec(shape):
    rank = len(shape)
    return pl.BlockSpec(shape, lambda i: (0,) * rank)


def _tc_forward(shared, oh, weights, interpret=False):
    in_specs = [
        pl.BlockSpec((BLK, IN_DIM), lambda i: (i, 0)),
        pl.BlockSpec((BLK, ND), lambda i: (i, 0)),
    ] + [_const_spec(w.shape) for w in weights]
    return pl.pallas_call(
        _tc_body,
        grid=(GRID,),
        in_specs=in_specs,
        out_specs=pl.BlockSpec((BLK, 1), lambda i: (i, 0)),
        out_shape=jax.ShapeDtypeStruct((B, 1), jnp.float32),
        compiler_params=pltpu.CompilerParams(
            dimension_semantics=("arbitrary",),
        ),
        interpret=interpret,
    )(shared, oh, *weights)


def _prep_weights(params):
    cw = params["cross_w"]
    cb = params["cross_b"]
    c1w, c1b = params["center"][0]
    c2w, c2b = params["center"][1]
    c3w, c3b = params["center"][2]
    f1w, f1b = params["final"][0]
    f2w, f2b = params["final"][1]
    a1w, a1b = params["aux"][0]
    a2w, a2b = params["aux"][1]
    row = lambda v: v.reshape(1, -1)
    return [
        params["dom_emb"],
        params["pn_w"], params["pn_b"],
        cw[:, :IN_DIM], cw[:, IN_DIM:],
        cb[:, :IN_DIM], cb[:, IN_DIM:],
        c1w[:IN_DIM], c1w[IN_DIM:], row(c1b),
        c2w, row(c2b), c3w, row(c3b),
        params["dW1"][:, :IN_DIM], params["dW1"][:, IN_DIM:], params["db1"],
        params["dW2"], params["db2"], params["dW3"], params["db3"],
        f1w, row(f1b), f2w, row(f2b),
        a1w, row(a1b), a2w, row(a2b),
    ]


def kernel(sparse_ids, domain_ids, params):
    ids3 = sparse_ids.reshape(NW, NCHUNK, CHUNK)
    table128 = _tc_transpose(params["emb"].T)
    table = table128.reshape(V_ROWS, D)
    rows = _sc_gather(table, ids3)
    shared = rows.reshape(B, IN_DIM)
    oh = jax.nn.one_hot(domain_ids, ND, dtype=jnp.float32)
    weights = _prep_weights(params)
    return _tc_forward(shared, oh, weights)
